# baseline (speedup 1.0000x reference)
import functools

import jax
import jax.numpy as jnp
from jax import lax
from jax.experimental import pallas as pl
from jax.experimental.pallas import tpu as pltpu
from jax.experimental.pallas import tpu_sc as plsc

N = 10000
E = 320000
D_IN = 128
D_OUT = 16
D = 2 * D_OUT

NC = 2
NS = 16
NW = NC * NS
B = 128
NB = E // B
KB = NB // NW
NXTRA = NB - KB * NW
NPAD = 10240
SLD = NPAD // NS
SLA = N // NS
NBUF = 6
LAG = 3
HBM_EVERY = 4

_mesh = plsc.VectorSubcoreMesh(core_axis_name="c", subcore_axis_name="s")
_params = pltpu.CompilerParams(use_tc_tiling_on_sc=False)


@functools.partial(
    pl.kernel,
    out_type=jax.ShapeDtypeStruct((NC, NPAD), jnp.float32),
    mesh=_mesh,
    compiler_params=_params,
    scratch_types=[
        pltpu.VMEM((KB + 1, B), jnp.int32),
        pltpu.VMEM((B,), jnp.float32),
        pltpu.VMEM_SHARED((NPAD,), jnp.float32),
        pltpu.SemaphoreType.DMA,
    ],
)
def _sc_degree(edge3_hbm, zeros1_hbm, deg_out, idx_v, ones_v, deg_sh, sem):
    c = lax.axis_index("c")
    s = lax.axis_index("s")
    wid = c * NS + s
    for i in range(B // 16):
        ones_v[pl.ds(i * 16, 16)] = jnp.full((16,), 1.0, jnp.float32)
    pltpu.sync_copy(edge3_hbm.at[1, pl.ds(wid * KB, KB)], idx_v.at[pl.ds(0, KB)])
    xrow = NW * KB + jnp.minimum(wid, NXTRA - 1)
    pltpu.sync_copy(edge3_hbm.at[1, pl.ds(xrow, 1)], idx_v.at[pl.ds(KB, 1)])
    pltpu.sync_copy(zeros1_hbm.at[pl.ds(s * SLD, SLD)],
                    deg_sh.at[pl.ds(s * SLD, SLD)])
    plsc.subcore_barrier()
    for g in range(0, KB, 8):
        descs = [
            pltpu.async_copy(ones_v, deg_sh.at[idx_v.at[k]], sem, add=True)
            for k in range(g, min(g + 8, KB))
        ]
        for d in descs:
            d.wait()

    @pl.when(wid < NXTRA)
    def _():
        pltpu.sync_copy(ones_v, deg_sh.at[idx_v.at[KB]], add=True)

    plsc.subcore_barrier()
    pltpu.sync_copy(deg_sh.at[pl.ds(s * SLD, SLD)],
                    deg_out.at[c, pl.ds(s * SLD, SLD)])


@functools.partial(
    pl.kernel,
    out_type=jax.ShapeDtypeStruct((NC, N, D), jnp.float32),
    mesh=_mesh,
    compiler_params=_params,
    scratch_types=[
        pltpu.VMEM((KB + 1, B), jnp.int32),
        pltpu.VMEM((KB + 1, B), jnp.int32),
        pltpu.VMEM((NBUF, B, D), jnp.float32),
        pltpu.VMEM_SHARED((N, D), jnp.float32),
        pltpu.VMEM_SHARED((N, D), jnp.float32),
        [pltpu.SemaphoreType.DMA] * NBUF,
        [pltpu.SemaphoreType.DMA] * NBUF,
    ],
)
def _sc_scatter(edge3_hbm, hs_hbm, zeros2_hbm, acc_out,
                rows_v, cols_v, bufs, acc_sh, hs_sh, gsems, ssems):
    c = lax.axis_index("c")
    s = lax.axis_index("s")
    wid = c * NS + s
    pltpu.sync_copy(edge3_hbm.at[0, pl.ds(wid * KB, KB)], rows_v.at[pl.ds(0, KB)])
    pltpu.sync_copy(edge3_hbm.at[1, pl.ds(wid * KB, KB)], cols_v.at[pl.ds(0, KB)])
    xrow = NW * KB + jnp.minimum(wid, NXTRA - 1)
    pltpu.sync_copy(edge3_hbm.at[0, pl.ds(xrow, 1)], rows_v.at[pl.ds(KB, 1)])
    pltpu.sync_copy(edge3_hbm.at[1, pl.ds(xrow, 1)], cols_v.at[pl.ds(KB, 1)])
    pltpu.sync_copy(hs_hbm.at[pl.ds(s * SLA, SLA)], hs_sh.at[pl.ds(s * SLA, SLA)])
    pltpu.sync_copy(zeros2_hbm.at[pl.ds(s * SLA, SLA)],
                    acc_sh.at[pl.ds(s * SLA, SLA)])
    plsc.subcore_barrier()

    gd = [None] * NBUF
    sd = [None] * NBUF
    for k in range(KB + LAG):
        if k < KB:
            b = k % NBUF
            if sd[b] is not None:
                sd[b].wait()
                sd[b] = None
            gd[b] = pltpu.async_copy(hs_sh.at[rows_v.at[k]], bufs.at[b], gsems[b])
        j = k - LAG
        if j >= 0:
            bj = j % NBUF
            gd[bj].wait()
            sd[bj] = pltpu.async_copy(
                bufs.at[bj], acc_sh.at[cols_v.at[j]], ssems[bj], add=True)
    for d in sd:
        if d is not None:
            d.wait()

    @pl.when(wid < NXTRA)
    def _():
        pltpu.async_copy(hs_sh.at[rows_v.at[KB]], bufs.at[0], gsems[0]).wait()
        pltpu.async_copy(bufs.at[0], acc_sh.at[cols_v.at[KB]], ssems[0],
                         add=True).wait()

    plsc.subcore_barrier()
    pltpu.sync_copy(acc_sh.at[pl.ds(s * SLA, SLA)],
                    acc_out.at[c, pl.ds(s * SLA, SLA)])


FSL0 = 313
FSL1 = 312
UNR = 8


@functools.partial(
    pl.kernel,
    out_type=(
        jax.ShapeDtypeStruct((N, D_OUT), jnp.float32),
        jax.ShapeDtypeStruct((N, D_OUT), jnp.float32),
        jax.ShapeDtypeStruct((N, D_OUT), jnp.float32),
    ),
    mesh=_mesh,
    compiler_params=pltpu.CompilerParams(use_tc_tiling_on_sc=False,
                                         needs_layout_passes=False),
    scratch_types=[
        pltpu.VMEM((FSL0, D), jnp.float32),
        pltpu.VMEM((FSL0, D), jnp.float32),
        pltpu.VMEM((FSL0, D), jnp.float32),
        pltpu.VMEM((FSL0, 1), jnp.float32),
        pltpu.VMEM((FSL0, D_OUT), jnp.float32),
        pltpu.VMEM((FSL0, D_OUT), jnp.float32),
    ],
)
def _sc_finish(acc_hbm, hs_hbm, s_hbm, mu_out, ls_out, zeta_out,
               va, vb, vh, vs, vmu, vls):
    c = lax.axis_index("c")
    s = lax.axis_index("s")

    def work(n0, nrows):
        pltpu.sync_copy(acc_hbm.at[0, pl.ds(n0, nrows)], va.at[pl.ds(0, nrows)])
        pltpu.sync_copy(acc_hbm.at[1, pl.ds(n0, nrows)], vb.at[pl.ds(0, nrows)])
        pltpu.sync_copy(hs_hbm.at[pl.ds(n0, nrows)], vh.at[pl.ds(0, nrows)])
        pltpu.sync_copy(s_hbm.at[pl.ds(n0, nrows)], vs.at[pl.ds(0, nrows)])

        zero16 = jnp.zeros((16,), jnp.int32)

        def one_row(r):
            sv = plsc.load_gather(vs, [jnp.full((16,), r, jnp.int32), zero16])
            t0 = (va[r, pl.ds(0, 16)] + vb[r, pl.ds(0, 16)]
                  + vh[r, pl.ds(0, 16)]) * sv
            t1 = (va[r, pl.ds(16, 16)] + vb[r, pl.ds(16, 16)]
                  + vh[r, pl.ds(16, 16)]) * sv
            vmu[r, pl.ds(0, 16)] = t0
            vls[r, pl.ds(0, 16)] = t1

        def body(g, carry):
            for dr in range(UNR):
                one_row(g * UNR + dr)
            return carry

        lax.fori_loop(0, nrows // UNR, body, 0)
        for r0 in range(nrows - nrows % UNR, nrows):
            one_row(r0)
        pltpu.sync_copy(vmu.at[pl.ds(0, nrows)], mu_out.at[pl.ds(n0, nrows)])
        pltpu.sync_copy(vls.at[pl.ds(0, nrows)], ls_out.at[pl.ds(n0, nrows)])
        pltpu.sync_copy(vmu.at[pl.ds(0, nrows)], zeta_out.at[pl.ds(n0, nrows)])

    @pl.when(c == 0)
    def _():
        work(s * FSL0, FSL0)

    @pl.when(c == 1)
    def _():
        work(NS * FSL0 + s * FSL1, FSL1)


def _tc_matmul_body(x_ref, w_ref, b_ref, h_ref):
    h = jnp.dot(x_ref[...], w_ref[...], preferred_element_type=jnp.float32)
    h_ref[...] = h + b_ref[...]


def _tc_scale_body(h_ref, degp_ref, hs_ref, s_ref):
    ones2 = jnp.ones((NC, 1), jnp.float32)
    deg = lax.dot_general(degp_ref[...], ones2, (((0,), (0,)), ((), ())),
                          preferred_element_type=jnp.float32) + 1.0
    sval = lax.rsqrt(deg)[:N]
    hs_ref[...] = h_ref[...] * sval
    s_ref[...] = sval


def kernel(x, edge_index, W_mu, b_mu, W_ls, b_ls):
    x = x.astype(jnp.float32)
    edge3 = edge_index.astype(jnp.int32).reshape(2, NB, B)

    Wcat = jnp.concatenate([W_mu, W_ls], axis=1).astype(jnp.float32)
    bcat = jnp.concatenate([b_mu, b_ls]).astype(jnp.float32)

    zeros1 = jnp.zeros((NPAD,), jnp.float32)
    zeros2 = jnp.zeros((N, D), jnp.float32)

    degp = _sc_degree(edge3, zeros1)

    h = pl.pallas_call(
        _tc_matmul_body,
        out_shape=jax.ShapeDtypeStruct((N, D), jnp.float32),
    )(x, Wcat, bcat)

    hs, s = pl.pallas_call(
        _tc_scale_body,
        out_shape=(
            jax.ShapeDtypeStruct((N, D), jnp.float32),
            jax.ShapeDtypeStruct((N, 1), jnp.float32),
        ),
    )(h, degp)

    acc = _sc_scatter(edge3, hs, zeros2)

    mu, ls, zeta = _sc_finish(acc, hs, s)

    return (mu, ls, zeta)


# speedup vs baseline: 1.0048x; 1.0048x over previous
import functools

import jax
import jax.numpy as jnp
from jax import lax
from jax.experimental import pallas as pl
from jax.experimental.pallas import tpu as pltpu
from jax.experimental.pallas import tpu_sc as plsc

N = 10000
E = 320000
D_IN = 128
D_OUT = 16
D = 2 * D_OUT

NC = 2
NS = 16
NW = NC * NS
B = 128
NB = E // B
KB = NB // NW
NXTRA = NB - KB * NW
NPAD = 10240
SLD = NPAD // NS
SLA = N // NS
NBUF = 6
LAG = 3

_mesh = plsc.VectorSubcoreMesh(core_axis_name="c", subcore_axis_name="s")
_params = pltpu.CompilerParams(use_tc_tiling_on_sc=False)


@functools.partial(
    pl.kernel,
    out_type=jax.ShapeDtypeStruct((NC, NPAD), jnp.float32),
    mesh=_mesh,
    compiler_params=_params,
    scratch_types=[
        pltpu.VMEM((KB + 1, B), jnp.int32),
        pltpu.VMEM((B,), jnp.float32),
        pltpu.VMEM_SHARED((NPAD,), jnp.float32),
        pltpu.SemaphoreType.DMA,
    ],
)
def _sc_degree(edge3_hbm, zeros1_hbm, deg_out, idx_v, ones_v, deg_sh, sem):
    c = lax.axis_index("c")
    s = lax.axis_index("s")
    wid = c * NS + s
    for i in range(B // 16):
        ones_v[pl.ds(i * 16, 16)] = jnp.full((16,), 1.0, jnp.float32)
    pltpu.sync_copy(edge3_hbm.at[1, pl.ds(wid * KB, KB)], idx_v.at[pl.ds(0, KB)])
    xrow = NW * KB + jnp.minimum(wid, NXTRA - 1)
    pltpu.sync_copy(edge3_hbm.at[1, pl.ds(xrow, 1)], idx_v.at[pl.ds(KB, 1)])
    pltpu.sync_copy(zeros1_hbm.at[pl.ds(s * SLD, SLD)],
                    deg_sh.at[pl.ds(s * SLD, SLD)])
    plsc.subcore_barrier()
    for g in range(0, KB, 8):
        descs = [
            pltpu.async_copy(ones_v, deg_sh.at[idx_v.at[k]], sem, add=True)
            for k in range(g, min(g + 8, KB))
        ]
        for d in descs:
            d.wait()

    @pl.when(wid < NXTRA)
    def _():
        pltpu.sync_copy(ones_v, deg_sh.at[idx_v.at[KB]], add=True)

    plsc.subcore_barrier()
    pltpu.sync_copy(deg_sh.at[pl.ds(s * SLD, SLD)],
                    deg_out.at[c, pl.ds(s * SLD, SLD)])


@functools.partial(
    pl.kernel,
    out_type=jax.ShapeDtypeStruct((NC, N, D), jnp.float32),
    mesh=_mesh,
    compiler_params=_params,
    scratch_types=[
        pltpu.VMEM((KB + 1, B), jnp.int32),
        pltpu.VMEM((KB + 1, B), jnp.int32),
        pltpu.VMEM((NBUF, B, D), jnp.float32),
        pltpu.VMEM_SHARED((N, D), jnp.float32),
        pltpu.VMEM_SHARED((N, D), jnp.float32),
        [pltpu.SemaphoreType.DMA] * NBUF,
        [pltpu.SemaphoreType.DMA] * NBUF,
    ],
)
def _sc_scatter(edge3_hbm, hs_hbm, zeros2_hbm, acc_out,
                rows_v, cols_v, bufs, acc_sh, hs_sh, gsems, ssems):
    c = lax.axis_index("c")
    s = lax.axis_index("s")
    wid = c * NS + s
    pltpu.sync_copy(edge3_hbm.at[0, pl.ds(wid * KB, KB)], rows_v.at[pl.ds(0, KB)])
    pltpu.sync_copy(edge3_hbm.at[1, pl.ds(wid * KB, KB)], cols_v.at[pl.ds(0, KB)])
    xrow = NW * KB + jnp.minimum(wid, NXTRA - 1)
    pltpu.sync_copy(edge3_hbm.at[0, pl.ds(xrow, 1)], rows_v.at[pl.ds(KB, 1)])
    pltpu.sync_copy(edge3_hbm.at[1, pl.ds(xrow, 1)], cols_v.at[pl.ds(KB, 1)])
    pltpu.sync_copy(hs_hbm.at[pl.ds(s * SLA, SLA)], hs_sh.at[pl.ds(s * SLA, SLA)])
    pltpu.sync_copy(zeros2_hbm.at[pl.ds(s * SLA, SLA)],
                    acc_sh.at[pl.ds(s * SLA, SLA)])
    plsc.subcore_barrier()

    gd = [None] * NBUF
    sd = [None] * NBUF
    for k in range(KB + LAG):
        if k < KB:
            b = k % NBUF
            if sd[b] is not None:
                sd[b].wait()
                sd[b] = None
            gd[b] = pltpu.async_copy(hs_sh.at[rows_v.at[k]], bufs.at[b], gsems[b])
        j = k - LAG
        if j >= 0:
            bj = j % NBUF
            gd[bj].wait()
            sd[bj] = pltpu.async_copy(
                bufs.at[bj], acc_sh.at[cols_v.at[j]], ssems[bj], add=True)
    for d in sd:
        if d is not None:
            d.wait()

    @pl.when(wid < NXTRA)
    def _():
        pltpu.async_copy(hs_sh.at[rows_v.at[KB]], bufs.at[0], gsems[0]).wait()
        pltpu.async_copy(bufs.at[0], acc_sh.at[cols_v.at[KB]], ssems[0],
                         add=True).wait()

    plsc.subcore_barrier()
    pltpu.sync_copy(acc_sh.at[pl.ds(s * SLA, SLA)],
                    acc_out.at[c, pl.ds(s * SLA, SLA)])


FSL0 = 313
FSL1 = 312
UNR = 8


@functools.partial(
    pl.kernel,
    out_type=(
        jax.ShapeDtypeStruct((N, D_OUT), jnp.float32),
        jax.ShapeDtypeStruct((N, D_OUT), jnp.float32),
        jax.ShapeDtypeStruct((N, D_OUT), jnp.float32),
    ),
    mesh=_mesh,
    compiler_params=pltpu.CompilerParams(use_tc_tiling_on_sc=False,
                                         needs_layout_passes=False),
    scratch_types=[
        pltpu.VMEM((FSL0, D), jnp.float32),
        pltpu.VMEM((FSL0, D), jnp.float32),
        pltpu.VMEM((FSL0, D), jnp.float32),
        pltpu.VMEM((FSL0, 1), jnp.float32),
        pltpu.VMEM((FSL0, D_OUT), jnp.float32),
        pltpu.VMEM((FSL0, D_OUT), jnp.float32),
    ],
)
def _sc_finish(acc_hbm, hs_hbm, s_hbm, mu_out, ls_out, zeta_out,
               va, vb, vh, vs, vmu, vls):
    c = lax.axis_index("c")
    s = lax.axis_index("s")

    def work(n0, nrows):
        pltpu.sync_copy(acc_hbm.at[0, pl.ds(n0, nrows)], va.at[pl.ds(0, nrows)])
        pltpu.sync_copy(acc_hbm.at[1, pl.ds(n0, nrows)], vb.at[pl.ds(0, nrows)])
        pltpu.sync_copy(hs_hbm.at[pl.ds(n0, nrows)], vh.at[pl.ds(0, nrows)])
        pltpu.sync_copy(s_hbm.at[pl.ds(n0, nrows)], vs.at[pl.ds(0, nrows)])

        zero16 = jnp.zeros((16,), jnp.int32)

        def one_row(r):
            sv = plsc.load_gather(vs, [jnp.full((16,), r, jnp.int32), zero16])
            t0 = (va[r, pl.ds(0, 16)] + vb[r, pl.ds(0, 16)]
                  + vh[r, pl.ds(0, 16)]) * sv
            t1 = (va[r, pl.ds(16, 16)] + vb[r, pl.ds(16, 16)]
                  + vh[r, pl.ds(16, 16)]) * sv
            vmu[r, pl.ds(0, 16)] = t0
            vls[r, pl.ds(0, 16)] = t1

        def body(g, carry):
            for dr in range(UNR):
                one_row(g * UNR + dr)
            return carry

        lax.fori_loop(0, nrows // UNR, body, 0)
        for r0 in range(nrows - nrows % UNR, nrows):
            one_row(r0)
        pltpu.sync_copy(vmu.at[pl.ds(0, nrows)], mu_out.at[pl.ds(n0, nrows)])
        pltpu.sync_copy(vls.at[pl.ds(0, nrows)], ls_out.at[pl.ds(n0, nrows)])
        pltpu.sync_copy(vmu.at[pl.ds(0, nrows)], zeta_out.at[pl.ds(n0, nrows)])

    @pl.when(c == 0)
    def _():
        work(s * FSL0, FSL0)

    @pl.when(c == 1)
    def _():
        work(NS * FSL0 + s * FSL1, FSL1)


def _tc_linear_body(x_ref, w_ref, b_ref, degp_ref, hs_ref, s_ref):
    ones2 = jnp.ones((NC, 1), jnp.float32)
    deg = lax.dot_general(degp_ref[...], ones2, (((0,), (0,)), ((), ())),
                          preferred_element_type=jnp.float32) + 1.0
    sval = lax.rsqrt(deg)[:N]
    h = jnp.dot(x_ref[...], w_ref[...], preferred_element_type=jnp.float32)
    hs_ref[...] = (h + b_ref[...]) * sval
    s_ref[...] = sval


def kernel(x, edge_index, W_mu, b_mu, W_ls, b_ls):
    x = x.astype(jnp.float32)
    edge3 = edge_index.astype(jnp.int32).reshape(2, NB, B)

    Wcat = jnp.concatenate([W_mu, W_ls], axis=1).astype(jnp.float32)
    bcat = jnp.concatenate([b_mu, b_ls]).astype(jnp.float32)

    zeros1 = jnp.zeros((NPAD,), jnp.float32)
    zeros2 = jnp.zeros((N, D), jnp.float32)

    degp = _sc_degree(edge3, zeros1)

    hs, s = pl.pallas_call(
        _tc_linear_body,
        out_shape=(
            jax.ShapeDtypeStruct((N, D), jnp.float32),
            jax.ShapeDtypeStruct((N, 1), jnp.float32),
        ),
    )(x, Wcat, bcat, degp)

    acc = _sc_scatter(edge3, hs, zeros2)

    mu, ls, zeta = _sc_finish(acc, hs, s)

    return (mu, ls, zeta)
